# Initial kernel scaffold; baseline (speedup 1.0000x reference)
import jax
import jax.numpy as jnp
from jax.experimental import pallas as pl


def kernel(x, edge_index, w1_0, b1_0, w2_0, b2_0, w1_1, b1_1, w2_1, b2_1, w1_2, b1_2, w2_2, b2_2, lin_w, lin_b):
    raise NotImplementedError("write your pallas kernel here")


# speedup vs baseline: 2.7930x; 2.7930x over previous
import functools

import jax
import jax.numpy as jnp
from jax import lax
from jax.experimental import pallas as pl
from jax.experimental.pallas import tpu as pltpu
from jax.experimental.pallas import tpu_sc as plsc

N, E, D, H, L, C = 10000, 320000, 128, 128, 3, 40

NC, NS = 2, 16
NW = NC * NS
CHUNK = 128
CHUNKS = 80
EPT = CHUNKS * CHUNK
ETOT = NW * EPT
NROWS = 10240
RPT = NROWS // NS
WB = RPT // CHUNK

_mesh = plsc.VectorSubcoreMesh(core_axis_name="c", subcore_axis_name="s")


@functools.partial(
    pl.kernel,
    out_type=(
        jax.ShapeDtypeStruct((NROWS, D), jnp.float32),
        jax.ShapeDtypeStruct((NROWS, D), jnp.float32),
    ),
    mesh=_mesh,
    scratch_types=(
        pltpu.VMEM((CHUNKS, CHUNK), jnp.int32),
        pltpu.VMEM((CHUNKS, CHUNK), jnp.int32),
        pltpu.VMEM((CHUNK, D), jnp.float32),
        pltpu.VMEM_SHARED((NROWS, D), jnp.float32),
        pltpu.SemaphoreType.DMA,
    ),
)
def _sc_agg(h_hbm, srcs_hbm, dsts_hbm, zeros_hbm, out0, out1,
            src_v, dst_v, rows_v, acc_sh, sem):
    c = lax.axis_index("c")
    s = lax.axis_index("s")
    wid = s * NC + c

    pltpu.sync_copy(srcs_hbm.at[wid], src_v)
    pltpu.sync_copy(dsts_hbm.at[wid], dst_v)
    pltpu.sync_copy(zeros_hbm, rows_v)

    base = s * RPT
    for k in range(WB):
        pltpu.sync_copy(rows_v, acc_sh.at[pl.ds(base + k * CHUNK, CHUNK)])
    plsc.subcore_barrier()

    def body(j, carry):
        pltpu.async_copy(h_hbm.at[src_v.at[j]], rows_v, sem).wait()
        pltpu.sync_copy(rows_v, acc_sh.at[dst_v.at[j]], add=True)
        return carry

    lax.fori_loop(0, CHUNKS, body, 0, unroll=False)
    plsc.subcore_barrier()

    def drain(out_ref):
        for k in range(WB):
            r0 = base + k * CHUNK
            pltpu.sync_copy(acc_sh.at[pl.ds(r0, CHUNK)], rows_v)
            pltpu.sync_copy(rows_v, out_ref.at[pl.ds(r0, CHUNK)])

    @pl.when(c == 0)
    def _():
        drain(out0)

    @pl.when(c == 1)
    def _():
        drain(out1)


def _gin_body(h_ref, a0_ref, a1_ref, w1_ref, b1_ref, w2_ref, b2_ref, o_ref):
    m = h_ref[...] + a0_ref[...] + a1_ref[...]
    t = jnp.dot(m, w1_ref[...], preferred_element_type=jnp.float32)
    t = jnp.maximum(t + b1_ref[...], 0.0)
    o_ref[...] = jnp.dot(t, w2_ref[...],
                         preferred_element_type=jnp.float32) + b2_ref[...]


_BN = 1024


def _gin_tc(h, a0, a1, w1, b1, w2, b2):
    wspec = pl.BlockSpec((D, H), lambda i: (0, 0))
    bspec = pl.BlockSpec((1, H), lambda i: (0, 0))
    return pl.pallas_call(
        _gin_body,
        grid=(NROWS // _BN,),
        in_specs=[
            pl.BlockSpec((_BN, D), lambda i: (i, 0)),
            pl.BlockSpec((_BN, D), lambda i: (i, 0)),
            pl.BlockSpec((_BN, D), lambda i: (i, 0)),
            wspec, bspec, wspec, bspec,
        ],
        out_specs=pl.BlockSpec((_BN, H), lambda i: (i, 0)),
        out_shape=jax.ShapeDtypeStruct((NROWS, H), jnp.float32),
    )(h, a0, a1, w1, b1, w2, b2)


_BF = 1000
_CAT = D + H * L


def _final_body(x_ref, h1_ref, h2_ref, h3_ref, w_ref, b_ref,
                cat_ref, pred_ref):
    cat = jnp.concatenate(
        [x_ref[...], h1_ref[...], h2_ref[...], h3_ref[...]], axis=-1)
    cat_ref[...] = cat
    pred_ref[...] = jnp.dot(cat, w_ref[...],
                            preferred_element_type=jnp.float32) + b_ref[...]


def _final_tc(x, h1, h2, h3, lin_w, lin_b):
    rspec = pl.BlockSpec((_BF, D), lambda i: (i, 0))
    return pl.pallas_call(
        _final_body,
        grid=(N // _BF,),
        in_specs=[
            rspec, rspec, rspec, rspec,
            pl.BlockSpec((_CAT, C), lambda i: (0, 0)),
            pl.BlockSpec((1, C), lambda i: (0, 0)),
        ],
        out_specs=[
            pl.BlockSpec((_BF, _CAT), lambda i: (i, 0)),
            pl.BlockSpec((_BF, C), lambda i: (i, 0)),
        ],
        out_shape=[
            jax.ShapeDtypeStruct((N, _CAT), jnp.float32),
            jax.ShapeDtypeStruct((N, C), jnp.float32),
        ],
    )(x, h1, h2, h3, lin_w, lin_b)


def kernel(x, edge_index, w1_0, b1_0, w2_0, b2_0, w1_1, b1_1, w2_1, b2_1,
           w1_2, b1_2, w2_2, b2_2, lin_w, lin_b):
    src = edge_index[0]
    dst = edge_index[1]
    pad = ETOT - E
    srcs = jnp.concatenate([src, jnp.zeros((pad,), jnp.int32)])
    dsts = jnp.concatenate([dst, jnp.full((pad,), N, jnp.int32)])
    srcs = srcs.reshape(NW, CHUNKS, CHUNK)
    dsts = dsts.reshape(NW, CHUNKS, CHUNK)

    xp = jnp.concatenate([x, jnp.zeros((NROWS - N, D), jnp.float32)])
    zeros128 = jnp.zeros((CHUNK, D), jnp.float32)

    layers = [(w1_0, b1_0, w2_0, b2_0), (w1_1, b1_1, w2_1, b2_1),
              (w1_2, b1_2, w2_2, b2_2)]
    h = xp
    hs = []
    for (w1, b1, w2, b2) in layers:
        a0, a1 = _sc_agg(h, srcs, dsts, zeros128)
        h = _gin_tc(h, a0, a1, w1, b1.reshape(1, H), w2, b2.reshape(1, H))
        hs.append(h)

    h_cat, pred = _final_tc(xp, hs[0], hs[1], hs[2],
                            lin_w, lin_b.reshape(1, C))
    return (pred, h_cat)
